# baseline (speedup 1.0000x reference)
import functools

import jax
import jax.numpy as jnp
from jax import lax
from jax.experimental import pallas as pl
from jax.experimental.pallas import tpu as pltpu
from jax.experimental.pallas import tpu_sc as plsc

_TIME_STEPS = 1000
_EMBED_DIM = 128
_BATCH = 16384

_NUM_CORES = 2
_NUM_SUBCORES = 16
_NUM_WORKERS = _NUM_CORES * _NUM_SUBCORES
_B_PER_W = _BATCH // _NUM_WORKERS
_CHUNK = 128
_CHUNKS_PER_W = _B_PER_W // _CHUNK


@functools.partial(
    pl.kernel,
    mesh=plsc.VectorSubcoreMesh(core_axis_name="c", subcore_axis_name="s"),
    out_type=jax.ShapeDtypeStruct((_BATCH, _EMBED_DIM), jnp.float32),
    scratch_types=[
        pltpu.VMEM((_CHUNKS_PER_W, _CHUNK), jnp.int32),
        pltpu.VMEM((_B_PER_W, _EMBED_DIM), jnp.float32),
        pltpu.SemaphoreType.DMA,
        pltpu.SemaphoreType.DMA,
        pltpu.SemaphoreType.DMA,
        pltpu.SemaphoreType.DMA,
        pltpu.SemaphoreType.DMA,
    ],
)
def _sc_gather(table_hbm, idx_hbm, out_hbm, idx_v, rows_v, s0, s1, s2, s3, ssem):
    gsems = [s0, s1, s2, s3]
    wid = lax.axis_index("s") * _NUM_CORES + lax.axis_index("c")
    base = wid * _B_PER_W
    pltpu.sync_copy(idx_hbm.at[pl.ds(wid * _CHUNKS_PER_W, _CHUNKS_PER_W)], idx_v)
    gathers = [
        pltpu.async_copy(
            table_hbm.at[idx_v.at[j]],
            rows_v.at[pl.ds(j * _CHUNK, _CHUNK)],
            gsems[j],
        )
        for j in range(_CHUNKS_PER_W)
    ]
    stores = []
    for j in range(_CHUNKS_PER_W):
        gathers[j].wait()
        stores.append(
            pltpu.async_copy(
                rows_v.at[pl.ds(j * _CHUNK, _CHUNK)],
                out_hbm.at[pl.ds(base + j * _CHUNK, _CHUNK)],
                ssem,
            )
        )
    for s in stores:
        s.wait()


def kernel(t, embeddings):
    idx2d = t.reshape(_NUM_WORKERS * _CHUNKS_PER_W, _CHUNK)
    out = _sc_gather(embeddings, idx2d)
    return out[:, :, None, None]


# speedup vs baseline: 1.1486x; 1.1486x over previous
import functools

import jax
import jax.numpy as jnp
from jax import lax
from jax.experimental import pallas as pl
from jax.experimental.pallas import tpu as pltpu
from jax.experimental.pallas import tpu_sc as plsc

_TIME_STEPS = 1000
_EMBED_DIM = 128
_BATCH = 16384

_NUM_CORES = 2
_NUM_SUBCORES = 16
_NUM_WORKERS = _NUM_CORES * _NUM_SUBCORES
_B_PER_W = _BATCH // _NUM_WORKERS
_CHUNK = 128
_CHUNKS_PER_W = _B_PER_W // _CHUNK


@functools.partial(
    pl.kernel,
    mesh=plsc.VectorSubcoreMesh(core_axis_name="c", subcore_axis_name="s"),
    out_type=jax.ShapeDtypeStruct((_BATCH, _EMBED_DIM), jnp.float32),
    scratch_types=[
        pltpu.VMEM_SHARED((_TIME_STEPS, _EMBED_DIM), jnp.float32),
        pltpu.VMEM((_CHUNKS_PER_W, _CHUNK), jnp.int32),
        pltpu.VMEM((_B_PER_W, _EMBED_DIM), jnp.float32),
        pltpu.SemaphoreType.DMA,
        pltpu.SemaphoreType.DMA,
        pltpu.SemaphoreType.DMA,
        pltpu.SemaphoreType.DMA,
        pltpu.SemaphoreType.DMA,
    ],
)
def _sc_gather(table_hbm, idx_hbm, out_hbm, table_sh, idx_v, rows_v,
               s0, s1, s2, s3, ssem):
    gsems = [s0, s1, s2, s3]
    sid = lax.axis_index("s")
    wid = sid * _NUM_CORES + lax.axis_index("c")
    base = wid * _B_PER_W
    @pl.when(sid == 0)
    def _():
        pltpu.sync_copy(table_hbm, table_sh)
    pltpu.sync_copy(idx_hbm.at[pl.ds(wid * _CHUNKS_PER_W, _CHUNKS_PER_W)], idx_v)
    plsc.subcore_barrier()
    gathers = [
        pltpu.async_copy(
            table_sh.at[idx_v.at[j]],
            rows_v.at[pl.ds(j * _CHUNK, _CHUNK)],
            gsems[j],
        )
        for j in range(_CHUNKS_PER_W)
    ]
    stores = []
    for j in range(_CHUNKS_PER_W):
        gathers[j].wait()
        stores.append(
            pltpu.async_copy(
                rows_v.at[pl.ds(j * _CHUNK, _CHUNK)],
                out_hbm.at[pl.ds(base + j * _CHUNK, _CHUNK)],
                ssem,
            )
        )
    for s in stores:
        s.wait()


def kernel(t, embeddings):
    idx2d = t.reshape(_NUM_WORKERS * _CHUNKS_PER_W, _CHUNK)
    out = _sc_gather(embeddings, idx2d)
    return out[:, :, None, None]


# speedup vs baseline: 1.1563x; 1.0067x over previous
import functools

import jax
import jax.numpy as jnp
from jax import lax
from jax.experimental import pallas as pl
from jax.experimental.pallas import tpu as pltpu
from jax.experimental.pallas import tpu_sc as plsc

_TIME_STEPS = 1000
_EMBED_DIM = 128
_BATCH = 16384

_NUM_CORES = 2
_NUM_SUBCORES = 16
_NUM_WORKERS = _NUM_CORES * _NUM_SUBCORES
_B_PER_W = _BATCH // _NUM_WORKERS
_CHUNK = 64
_CHUNKS_PER_W = _B_PER_W // _CHUNK
_BCAST_ROWS = 128
_BCAST_FULL = _TIME_STEPS // _BCAST_ROWS
_BCAST_REST = _TIME_STEPS - _BCAST_FULL * _BCAST_ROWS


@functools.partial(
    pl.kernel,
    mesh=plsc.VectorSubcoreMesh(core_axis_name="c", subcore_axis_name="s"),
    out_type=jax.ShapeDtypeStruct((_BATCH, _EMBED_DIM), jnp.float32),
    scratch_types=[
        pltpu.VMEM_SHARED((_TIME_STEPS, _EMBED_DIM), jnp.float32),
        pltpu.VMEM((_CHUNKS_PER_W, _CHUNK), jnp.int32),
        pltpu.VMEM((_B_PER_W, _EMBED_DIM), jnp.float32),
    ]
    + [pltpu.SemaphoreType.DMA] * (_CHUNKS_PER_W + 1),
)
def _sc_gather(table_hbm, idx_hbm, out_hbm, table_sh, idx_v, rows_v, *sems):
    gsems, ssem = sems[:_CHUNKS_PER_W], sems[_CHUNKS_PER_W]
    sid = lax.axis_index("s")
    wid = sid * _NUM_CORES + lax.axis_index("c")
    base = wid * _B_PER_W
    @pl.when(sid < _BCAST_FULL)
    def _():
        pltpu.sync_copy(
            table_hbm.at[pl.ds(sid * _BCAST_ROWS, _BCAST_ROWS)],
            table_sh.at[pl.ds(sid * _BCAST_ROWS, _BCAST_ROWS)],
        )
    @pl.when(sid == _BCAST_FULL)
    def _():
        pltpu.sync_copy(
            table_hbm.at[pl.ds(_BCAST_FULL * _BCAST_ROWS, _BCAST_REST)],
            table_sh.at[pl.ds(_BCAST_FULL * _BCAST_ROWS, _BCAST_REST)],
        )
    pltpu.sync_copy(idx_hbm.at[pl.ds(wid * _CHUNKS_PER_W, _CHUNKS_PER_W)], idx_v)
    plsc.subcore_barrier()
    gathers = [
        pltpu.async_copy(
            table_sh.at[idx_v.at[j]],
            rows_v.at[pl.ds(j * _CHUNK, _CHUNK)],
            gsems[j],
        )
        for j in range(_CHUNKS_PER_W)
    ]
    stores = []
    for j in range(_CHUNKS_PER_W):
        gathers[j].wait()
        stores.append(
            pltpu.async_copy(
                rows_v.at[pl.ds(j * _CHUNK, _CHUNK)],
                out_hbm.at[pl.ds(base + j * _CHUNK, _CHUNK)],
                ssem,
            )
        )
    for s in stores:
        s.wait()


def kernel(t, embeddings):
    idx2d = t.reshape(_NUM_WORKERS * _CHUNKS_PER_W, _CHUNK)
    out = _sc_gather(embeddings, idx2d)
    return out[:, :, None, None]
